# baseline (speedup 1.0000x reference)
import numpy as np
import jax
import jax.numpy as jnp
from jax.experimental import pallas as pl
from jax.experimental.pallas import tpu as pltpu

TOPK = 100
SBLK = 512

_I32_MIN = np.int32(-2**31)
_BITMASKS = [np.int32(np.uint32(1 << b)) for b in range(31, -1, -1)]


def _bf(x):
    return x.astype(jnp.bfloat16)


def _bfr(x):
    return x.astype(jnp.bfloat16).astype(jnp.float32)


def _score_kernel(q_ref, v_ref, w1_ref, w2_ref, b1_ref, b2_ref, vv_ref,
                  bv_ref, wo_ref, bo_ref, score_ref):
    nb = q_ref.shape[0]
    u = vv_ref.shape[1]
    k = vv_ref.shape[0]
    qb = jnp.dot(_bf(q_ref[...]), _bf(w2_ref[...]),
                 preferred_element_type=jnp.float32) + b2_ref[...]
    vvb = _bfr(vv_ref[...])
    for b in range(nb):
        s1 = jnp.dot(_bf(v_ref[b]), _bf(w1_ref[...]),
                     preferred_element_type=jnp.float32)
        s1 = (s1 + b1_ref[...]) + qb[b][None, :]
        t = _bfr(jnp.tanh(s1))
        acc = jnp.zeros((t.shape[0],), dtype=jnp.float32)
        for h in range(k):
            seg = t[:, h * u:(h + 1) * u] * vvb[h][None, :]
            sc_h = jnp.sum(seg, axis=1) + bv_ref[0, h]
            acc = acc + _bfr(sc_h) * _bfr(wo_ref[0, h])
        score_ref[b, :] = acc + bo_ref[0, 0]


def _radix_kth_largest(ikey, kvec):
    c = jnp.zeros((ikey.shape[0], 1), dtype=jnp.int32)
    for m in _BITMASKS:
        ctry = c | m
        sctry = ctry ^ _I32_MIN
        cnt = jnp.sum((ikey >= sctry).astype(jnp.int32), axis=1, keepdims=True)
        c = jnp.where(cnt >= kvec, ctry, c)
    return c ^ _I32_MIN


def _topk_kernel(score_ref, ms_ref, aw_ref):
    s = score_ref[...]
    nb, ns = s.shape
    bits = jax.lax.bitcast_convert_type(s, jnp.int32)
    ikey = jnp.where(bits >= 0, bits, bits ^ np.int32(0x7FFFFFFF))
    kvec = jnp.full((nb, 1), TOPK, dtype=jnp.int32)
    kth = _radix_kth_largest(ikey, kvec)
    gt = ikey > kth
    eq = ikey == kth
    n_gt = jnp.sum(gt.astype(jnp.int32), axis=1, keepdims=True)
    needed = kvec - n_gt
    iota = jax.lax.broadcasted_iota(jnp.int32, (nb, ns), 1)
    key2 = jnp.where(eq, -iota, _I32_MIN)
    idx_cut = -_radix_kth_largest(key2, needed)
    mask = gt | (eq & (iota <= idx_cut))
    ms = jnp.where(mask, s, 0.0)
    sig = jax.nn.sigmoid(ms)
    colsum = jnp.sum(sig, axis=0, keepdims=True)
    ms_ref[...] = ms
    aw_ref[...] = sig / colsum


def _ctx_kernel(aw_ref, v_ref, ctx_ref):
    j = pl.program_id(0)

    @pl.when(j == 0)
    def _():
        ctx_ref[...] = jnp.zeros_like(ctx_ref)

    nb = aw_ref.shape[0]
    for b in range(nb):
        ctx_ref[b, :] = ctx_ref[b, :] + jnp.dot(
            aw_ref[b], v_ref[b], preferred_element_type=jnp.float32)


def kernel(query, values, W1, b1, W2, b2, V, bV, Wo, bo):
    Bn, Sn, Hn = values.shape
    Kn, _, Un = W1.shape
    KU = Kn * Un
    nj = Sn // SBLK

    w1c = jnp.transpose(W1, (1, 0, 2)).reshape(Hn, KU)
    w2c = jnp.transpose(W2, (1, 0, 2)).reshape(Hn, KU)
    b1c = b1.reshape(1, KU)
    b2c = b2.reshape(1, KU)
    bvr = bV.reshape(1, Kn)
    wor = Wo.reshape(1, Kn)
    bor = bo.reshape(1, 1)

    score = pl.pallas_call(
        _score_kernel,
        grid=(nj,),
        in_specs=[
            pl.BlockSpec((Bn, Hn), lambda j: (0, 0)),
            pl.BlockSpec((Bn, SBLK, Hn), lambda j: (0, j, 0)),
            pl.BlockSpec((Hn, KU), lambda j: (0, 0)),
            pl.BlockSpec((Hn, KU), lambda j: (0, 0)),
            pl.BlockSpec((1, KU), lambda j: (0, 0)),
            pl.BlockSpec((1, KU), lambda j: (0, 0)),
            pl.BlockSpec((Kn, Un), lambda j: (0, 0)),
            pl.BlockSpec((1, Kn), lambda j: (0, 0)),
            pl.BlockSpec((1, Kn), lambda j: (0, 0)),
            pl.BlockSpec((1, 1), lambda j: (0, 0)),
        ],
        out_specs=pl.BlockSpec((Bn, SBLK), lambda j: (0, j)),
        out_shape=jax.ShapeDtypeStruct((Bn, Sn), jnp.float32),
    )(query, values, w1c, w2c, b1c, b2c, V, bvr, wor, bor)

    ms, aw = pl.pallas_call(
        _topk_kernel,
        out_shape=(jax.ShapeDtypeStruct((Bn, Sn), jnp.float32),
                   jax.ShapeDtypeStruct((Bn, Sn), jnp.float32)),
    )(score)

    ctx = pl.pallas_call(
        _ctx_kernel,
        grid=(nj,),
        in_specs=[
            pl.BlockSpec((Bn, SBLK), lambda j: (0, j)),
            pl.BlockSpec((Bn, SBLK, Hn), lambda j: (0, j, 0)),
        ],
        out_specs=pl.BlockSpec((Bn, Hn), lambda j: (0, 0)),
        out_shape=jax.ShapeDtypeStruct((Bn, Hn), jnp.float32),
    )(aw, values)

    return (ctx, aw[..., None], ms[..., None])


# speedup vs baseline: 1.7424x; 1.7424x over previous
import numpy as np
import jax
import jax.numpy as jnp
from jax.experimental import pallas as pl
from jax.experimental.pallas import tpu as pltpu

TOPK = 100
SBLK = 512

_I32_MIN = np.int32(-2**31)
_BITMASKS = [np.int32(np.uint32(1 << b)) for b in range(31, -1, -1)]


def _score_kernel(q_ref, v_ref, w1t_ref, qb_ref, vblk_ref, wo_ref, c0_ref,
                  score_ref):
    nb = q_ref.shape[0]
    dn = (((1,), (1,)), ((), ()))
    for b in range(nb):
        s1t = jax.lax.dot_general(w1t_ref[...], v_ref[b], dn,
                                  precision=jax.lax.Precision.DEFAULT,
                                  preferred_element_type=jnp.float32)
        t = jnp.tanh(s1t + qb_ref[:, b][:, None])
        tb = t.astype(jnp.bfloat16)
        sct = jax.lax.dot_general(vblk_ref[...], tb, (((1,), (0,)), ((), ())),
                                  preferred_element_type=jnp.float32)
        score_ref[b, :] = jnp.sum(sct * wo_ref[...], axis=0) + c0_ref[0, 0]


def _radix_kth_largest(ikey, kvec):
    c = jnp.zeros((ikey.shape[0], 1), dtype=jnp.int32)
    for m in _BITMASKS:
        ctry = c | m
        sctry = ctry ^ _I32_MIN
        cnt = jnp.sum((ikey >= sctry).astype(jnp.int32), axis=1, keepdims=True)
        c = jnp.where(cnt >= kvec, ctry, c)
    return c ^ _I32_MIN


def _topk_kernel(score_ref, ms_ref, aw_ref):
    s = score_ref[...]
    nb, ns = s.shape
    bits = jax.lax.bitcast_convert_type(s, jnp.int32)
    ikey = jnp.where(bits >= 0, bits, bits ^ np.int32(0x7FFFFFFF))
    kvec = jnp.full((nb, 1), TOPK, dtype=jnp.int32)
    kth = _radix_kth_largest(ikey, kvec)
    gt = ikey > kth
    eq = ikey == kth
    n_gt = jnp.sum(gt.astype(jnp.int32), axis=1, keepdims=True)
    needed = kvec - n_gt
    iota = jax.lax.broadcasted_iota(jnp.int32, (nb, ns), 1)
    key2 = jnp.where(eq, -iota, _I32_MIN)
    idx_cut = -_radix_kth_largest(key2, needed)
    mask = gt | (eq & (iota <= idx_cut))
    ms = jnp.where(mask, s, 0.0)
    sig = jax.nn.sigmoid(ms)
    colsum = jnp.sum(sig, axis=0, keepdims=True)
    ms_ref[...] = ms
    aw_ref[...] = sig / colsum


def _ctx_kernel(aw_ref, v_ref, ctx_ref):
    j = pl.program_id(0)

    @pl.when(j == 0)
    def _():
        ctx_ref[...] = jnp.zeros_like(ctx_ref)

    nb = aw_ref.shape[0]
    for b in range(nb):
        ctx_ref[b, :] = ctx_ref[b, :] + jnp.dot(
            aw_ref[b], v_ref[b], precision=jax.lax.Precision.HIGHEST,
            preferred_element_type=jnp.float32)


def kernel(query, values, W1, b1, W2, b2, V, bV, Wo, bo):
    Bn, Sn, Hn = values.shape
    Kn, _, Un = W1.shape
    KU = Kn * Un
    nj = Sn // SBLK

    w1t = jnp.transpose(W1, (0, 2, 1)).reshape(KU, Hn)
    qb = jnp.concatenate(
        [query @ W2[i] + b2[i] + b1[i] for i in range(Kn)], axis=1)
    qbt = qb.T
    rows = np.repeat(np.arange(Kn), Un)
    cols = np.arange(KU)
    vblk = jnp.zeros((Kn, KU), jnp.float32).at[rows, cols].set(
        V.reshape(KU)).astype(jnp.bfloat16)
    wo = Wo.reshape(Kn, 1)
    c0 = (jnp.dot(bV, Wo) + bo).reshape(1, 1)

    score = pl.pallas_call(
        _score_kernel,
        grid=(nj,),
        in_specs=[
            pl.BlockSpec((Bn, Hn), lambda j: (0, 0)),
            pl.BlockSpec((Bn, SBLK, Hn), lambda j: (0, j, 0)),
            pl.BlockSpec((KU, Hn), lambda j: (0, 0)),
            pl.BlockSpec((KU, Bn), lambda j: (0, 0)),
            pl.BlockSpec((Kn, KU), lambda j: (0, 0)),
            pl.BlockSpec((Kn, 1), lambda j: (0, 0)),
            pl.BlockSpec((1, 1), lambda j: (0, 0)),
        ],
        out_specs=pl.BlockSpec((Bn, SBLK), lambda j: (0, j)),
        out_shape=jax.ShapeDtypeStruct((Bn, Sn), jnp.float32),
    )(query, values, w1t, qbt, vblk, wo, c0)

    ms, aw = pl.pallas_call(
        _topk_kernel,
        out_shape=(jax.ShapeDtypeStruct((Bn, Sn), jnp.float32),
                   jax.ShapeDtypeStruct((Bn, Sn), jnp.float32)),
    )(score)

    ctx = pl.pallas_call(
        _ctx_kernel,
        grid=(nj,),
        in_specs=[
            pl.BlockSpec((Bn, SBLK), lambda j: (0, j)),
            pl.BlockSpec((Bn, SBLK, Hn), lambda j: (0, j, 0)),
        ],
        out_specs=pl.BlockSpec((Bn, Hn), lambda j: (0, 0)),
        out_shape=jax.ShapeDtypeStruct((Bn, Hn), jnp.float32),
    )(aw, values)

    return (ctx, aw[..., None], ms[..., None])
